# Initial kernel scaffold; baseline (speedup 1.0000x reference)
import jax
import jax.numpy as jnp
from jax.experimental import pallas as pl


def kernel(x, weights, input_indices, output_indices, bias):
    raise NotImplementedError("write your pallas kernel here")


# speedup vs baseline: 8.9659x; 8.9659x over previous
import functools

import jax
import jax.numpy as jnp
from jax.experimental import pallas as pl
from jax.experimental.pallas import tpu as pltpu


_G = 4


def _a1_body(wt_ref, idxo_ref, tmp_ref, *, bout, bin_, d_out, qt):
    s = _G * bout
    idx_out = idxo_ref[0, 0, :]
    parts = []
    for sb in range(_G):
        w_sb = wt_ref[sb]
        parts.append(jnp.pad(w_sb, ((0, 0), (sb * bout, s - (sb + 1) * bout))))
    bd = jnp.concatenate(parts, axis=0)
    for q0 in range(0, d_out, qt):
        cols = jax.lax.broadcasted_iota(jnp.int32, (s, qt), 1) + q0
        oh = (idx_out[:, None] == cols).astype(jnp.bfloat16)
        tmp_ref[:, q0:q0 + qt] = jax.lax.dot_general(
            bd, oh, (((1,), (0,)), ((), ())),
            preferred_element_type=jnp.float32).astype(jnp.bfloat16)


def _a2_body(idxi_ref, tmp_ref, m_ref, *, pt, qw, n_slots, kc):
    p0 = pl.program_id(1) * pt
    acc = jnp.zeros((pt, qw), jnp.float32)
    for k0 in range(0, n_slots, kc):
        idx_sl = idxi_ref[0, k0:k0 + kc]
        rows = jax.lax.broadcasted_iota(jnp.int32, (pt, kc), 0) + p0
        oh = (rows == idx_sl[None, :]).astype(jnp.bfloat16)
        acc = acc + jax.lax.dot_general(
            oh, tmp_ref[k0:k0 + kc, :], (((1,), (0,)), ((), ())),
            preferred_element_type=jnp.float32)
    m_ref[...] = acc.astype(jnp.bfloat16)


def _b_body(x_ref, m_ref, b_ref, o_ref, *, qt, kc):
    xv = x_ref[...]
    tt, d_in = xv.shape
    qw = o_ref.shape[1]
    for q0 in range(0, qw, qt):
        acc = jnp.zeros((tt, qt), jnp.float32)
        for k0 in range(0, d_in, kc):
            acc = acc + jax.lax.dot_general(
                xv[:, k0:k0 + kc], m_ref[k0:k0 + kc, q0:q0 + qt],
                (((1,), (0,)), ((), ())),
                preferred_element_type=jnp.float32)
        o_ref[:, q0:q0 + qt] = acc + b_ref[0, q0:q0 + qt][None, :]


def kernel(x, weights, input_indices, output_indices, bias):
    t, d_in = x.shape
    nb, bout, bin_ = weights.shape
    d_out = bias.shape[0]
    ns = nb * bin_

    wt = weights.transpose(0, 2, 1).astype(jnp.bfloat16)
    idxo = output_indices.reshape(nb // _G, 1, _G * bout)
    idxi = input_indices.reshape(1, ns)
    xb = x.astype(jnp.bfloat16)
    bias2 = bias.reshape(1, d_out)

    tmp = pl.pallas_call(
        functools.partial(_a1_body, bout=bout, bin_=bin_, d_out=d_out,
                          qt=min(512, d_out)),
        grid=(nb // _G,),
        in_specs=[
            pl.BlockSpec((_G, bin_, bout), lambda g: (g, 0, 0)),
            pl.BlockSpec((1, 1, _G * bout), lambda g: (g, 0, 0)),
        ],
        out_specs=pl.BlockSpec((_G * bin_, d_out), lambda g: (g, 0)),
        out_shape=jax.ShapeDtypeStruct((ns, d_out), jnp.bfloat16),
        compiler_params=pltpu.CompilerParams(
            dimension_semantics=("parallel",)),
    )(wt, idxo)

    ct, ptile = 8, 256
    qw = d_out // ct
    m = pl.pallas_call(
        functools.partial(_a2_body, pt=ptile, qw=qw, n_slots=ns,
                          kc=min(2048, ns)),
        grid=(ct, d_in // ptile),
        in_specs=[
            pl.BlockSpec((1, ns), lambda c, p: (0, 0)),
            pl.BlockSpec((ns, qw), lambda c, p: (0, c)),
        ],
        out_specs=pl.BlockSpec((ptile, qw), lambda c, p: (p, c)),
        out_shape=jax.ShapeDtypeStruct((d_in, d_out), jnp.bfloat16),
        compiler_params=pltpu.CompilerParams(
            dimension_semantics=("arbitrary", "arbitrary")),
    )(idxi, tmp)

    qsplit, ttile = 2, 256
    out = pl.pallas_call(
        functools.partial(_b_body, qt=min(512, d_out // qsplit),
                          kc=min(2048, d_in)),
        grid=(qsplit, t // ttile),
        in_specs=[
            pl.BlockSpec((ttile, d_in), lambda q, i: (i, 0)),
            pl.BlockSpec((d_in, d_out // qsplit), lambda q, i: (0, q)),
            pl.BlockSpec((1, d_out // qsplit), lambda q, i: (0, q)),
        ],
        out_specs=pl.BlockSpec((ttile, d_out // qsplit), lambda q, i: (i, q)),
        out_shape=jax.ShapeDtypeStruct((t, d_out), jnp.float32),
        compiler_params=pltpu.CompilerParams(
            dimension_semantics=("arbitrary", "arbitrary")),
    )(xb, m, bias2)
    return out


# speedup vs baseline: 8.9705x; 1.0005x over previous
import functools

import jax
import jax.numpy as jnp
from jax.experimental import pallas as pl
from jax.experimental.pallas import tpu as pltpu


_G = 4


def _a1_body(wt_ref, idxo_ref, tmp_ref, *, bout, bin_, d_out, qt):
    s = _G * bout
    idx_out = idxo_ref[0, 0, :]
    parts = []
    for sb in range(_G):
        w_sb = wt_ref[sb]
        parts.append(jnp.pad(w_sb, ((0, 0), (sb * bout, s - (sb + 1) * bout))))
    bd = jnp.concatenate(parts, axis=0)
    for q0 in range(0, d_out, qt):
        cols = jax.lax.broadcasted_iota(jnp.int32, (s, qt), 1) + q0
        oh = (idx_out[:, None] == cols).astype(jnp.bfloat16)
        tmp_ref[:, q0:q0 + qt] = jax.lax.dot_general(
            bd, oh, (((1,), (0,)), ((), ())),
            preferred_element_type=jnp.float32).astype(jnp.bfloat16)


def _a2_body(idxi_ref, tmp_ref, m_ref, *, pt, qw, n_slots, kc):
    p0 = pl.program_id(1) * pt
    acc = jnp.zeros((pt, qw), jnp.float32)
    for k0 in range(0, n_slots, kc):
        idx_sl = idxi_ref[0, k0:k0 + kc]
        rows = jax.lax.broadcasted_iota(jnp.int32, (pt, kc), 0) + p0
        oh = (rows == idx_sl[None, :]).astype(jnp.bfloat16)
        acc = acc + jax.lax.dot_general(
            oh, tmp_ref[k0:k0 + kc, :], (((1,), (0,)), ((), ())),
            preferred_element_type=jnp.float32)
    m_ref[...] = acc.astype(jnp.bfloat16)


def _b_body(x_ref, m_ref, b_ref, o_ref, *, qt, kc):
    xv = x_ref[...]
    tt, d_in = xv.shape
    qw = o_ref.shape[1]
    for q0 in range(0, qw, qt):
        acc = jnp.zeros((tt, qt), jnp.float32)
        for k0 in range(0, d_in, kc):
            acc = acc + jax.lax.dot_general(
                xv[:, k0:k0 + kc], m_ref[k0:k0 + kc, q0:q0 + qt],
                (((1,), (0,)), ((), ())),
                preferred_element_type=jnp.float32)
        o_ref[:, q0:q0 + qt] = acc + b_ref[0, q0:q0 + qt][None, :]


def kernel(x, weights, input_indices, output_indices, bias):
    t, d_in = x.shape
    nb, bout, bin_ = weights.shape
    d_out = bias.shape[0]
    ns = nb * bin_

    wt = weights.transpose(0, 2, 1).astype(jnp.bfloat16)
    idxo = output_indices.reshape(nb // _G, 1, _G * bout)
    idxi = input_indices.reshape(1, ns)
    xb = x.astype(jnp.bfloat16)
    bias2 = bias.reshape(1, d_out)

    tmp = pl.pallas_call(
        functools.partial(_a1_body, bout=bout, bin_=bin_, d_out=d_out,
                          qt=min(512, d_out)),
        grid=(nb // _G,),
        in_specs=[
            pl.BlockSpec((_G, bin_, bout), lambda g: (g, 0, 0)),
            pl.BlockSpec((1, 1, _G * bout), lambda g: (g, 0, 0)),
        ],
        out_specs=pl.BlockSpec((_G * bin_, d_out), lambda g: (g, 0)),
        out_shape=jax.ShapeDtypeStruct((ns, d_out), jnp.bfloat16),
        compiler_params=pltpu.CompilerParams(
            dimension_semantics=("parallel",)),
    )(wt, idxo)

    ct, ptile = 8, 256
    qw = d_out // ct
    m = pl.pallas_call(
        functools.partial(_a2_body, pt=ptile, qw=qw, n_slots=ns,
                          kc=min(2048, ns)),
        grid=(ct, d_in // ptile),
        in_specs=[
            pl.BlockSpec((1, ns), lambda c, p: (0, 0)),
            pl.BlockSpec((ns, qw), lambda c, p: (0, c)),
        ],
        out_specs=pl.BlockSpec((ptile, qw), lambda c, p: (p, c)),
        out_shape=jax.ShapeDtypeStruct((d_in, d_out), jnp.bfloat16),
        compiler_params=pltpu.CompilerParams(
            dimension_semantics=("parallel", "parallel")),
    )(idxi, tmp)

    qsplit, ttile = 2, 256
    out = pl.pallas_call(
        functools.partial(_b_body, qt=min(512, d_out // qsplit),
                          kc=min(2048, d_in)),
        grid=(qsplit, t // ttile),
        in_specs=[
            pl.BlockSpec((ttile, d_in), lambda q, i: (i, 0)),
            pl.BlockSpec((d_in, d_out // qsplit), lambda q, i: (0, q)),
            pl.BlockSpec((1, d_out // qsplit), lambda q, i: (0, q)),
        ],
        out_specs=pl.BlockSpec((ttile, d_out // qsplit), lambda q, i: (i, q)),
        out_shape=jax.ShapeDtypeStruct((t, d_out), jnp.float32),
        compiler_params=pltpu.CompilerParams(
            dimension_semantics=("parallel", "parallel")),
    )(xb, m, bias2)
    return out
